# baseline (speedup 1.0000x reference)
import jax
import jax.numpy as jnp
from jax import lax
from jax.experimental import pallas as pl
from jax.experimental.pallas import tpu as pltpu
from jax.experimental.pallas import tpu_sc as plsc

N_ROWS = 320000
D = 128
NSEG = 1024
NC = 2
NS = 16
NW = NC * NS
T = 256
C = 128
NCH = T // C
NT = N_ROWS // T
TPW = (NT + NW - 1) // NW
LANES = 16
SEG_PER_W = NSEG // NS


def _body(x_hbm, b_hbm, sums_hbm, cnts_hbm, xbuf0, xbuf1, idv0, idv1,
          zbuf, lcnt, acc_sh, sx0, sx1, si0, si1):
    cid = lax.axis_index("c")
    sid = lax.axis_index("s")
    wid = cid * NS + sid

    zf = jnp.zeros((LANES,), jnp.float32)
    for r in range(SEG_PER_W):
        for j in range(D // LANES):
            zbuf[r, pl.ds(j * LANES, LANES)] = zf
    for r in range(NSEG):
        lcnt[pl.ds(r * LANES, LANES)] = zf
    my0 = sid * SEG_PER_W
    pltpu.sync_copy(zbuf, acc_sh.at[pl.ds(my0, SEG_PER_W), :])
    plsc.subcore_barrier()

    nt = lax.div(NT - 1 - wid, NW) + 1

    xbufs = (xbuf0, xbuf1)
    idvs = (idv0, idv1)
    sxs = (sx0, sx1)
    sis = (si0, si1)

    def start(t, b):
        base = pl.multiple_of((t * NW + wid) * T, T)
        pltpu.async_copy(x_hbm.at[pl.ds(base, T), :], xbufs[b], sxs[b])
        for ch in range(NCH):
            pltpu.async_copy(b_hbm.at[pl.ds(base + ch * C, C)],
                             idvs[b].at[ch], sis[b])

    def wait(b):
        pltpu.make_async_copy(x_hbm.at[pl.ds(0, T), :], xbufs[b],
                              sxs[b]).wait()
        for ch in range(NCH):
            pltpu.make_async_copy(b_hbm.at[pl.ds(0, C)], idvs[b].at[ch],
                                  sis[b]).wait()

    ones16 = jnp.full((LANES,), 1.0, jnp.float32)
    full16 = jnp.full((LANES,), float(LANES), jnp.float32)

    def process(b):
        for ch in range(NCH):
            pltpu.sync_copy(xbufs[b].at[pl.ds(ch * C, C), :],
                            acc_sh.at[idvs[b].at[ch]], add=True)
        for ch in range(NCH):
            for g in range(C // LANES):
                iv = idvs[b][ch, pl.ds(g * LANES, LANES)]
                i0 = iv[0]
                i15 = iv[LANES - 1]
                uniform = i0 == i15

                @pl.when(uniform)
                def _():
                    plsc.addupdate(lcnt.at[pl.ds(i0 * LANES, LANES)],
                                   full16)

                @pl.when(jnp.logical_not(uniform))
                def _():
                    for kk in range(LANES):
                        plsc.addupdate(
                            lcnt.at[pl.ds(iv[kk] * LANES, LANES)], ones16)

    @pl.when(nt > 0)
    def _():
        start(0, 0)

    @pl.when(nt > 1)
    def _():
        start(1, 1)

    def pair_body(p, _):
        for b in range(2):
            t = p * 2 + b

            @pl.when(t < nt)
            def _():
                wait(b)
                process(b)

                @pl.when(t + 2 < nt)
                def _():
                    start(t + 2, b)

        return 0

    lax.fori_loop(0, lax.div(nt + 1, 2), pair_body, 0)

    plsc.subcore_barrier()
    out_row = cid * NSEG + my0
    pltpu.sync_copy(acc_sh.at[pl.ds(my0, SEG_PER_W), :], zbuf)
    pltpu.sync_copy(zbuf, sums_hbm.at[pl.ds(out_row, SEG_PER_W), :])
    pltpu.sync_copy(lcnt, cnts_hbm.at[pl.ds(wid * NSEG * LANES,
                                            NSEG * LANES)])


def _combine_body(s_ref, c_ref, o_ref):
    s = s_ref[0] + s_ref[1]
    c = jnp.maximum(jnp.sum(c_ref[...], axis=0), 1.0)
    o_ref[...] = s / c[:, :1]


@jax.jit
def _pooled(x, batch):
    mesh = plsc.VectorSubcoreMesh(core_axis_name="c", subcore_axis_name="s")
    f = pl.kernel(
        _body,
        out_type=(
            jax.ShapeDtypeStruct((NC * NSEG, D), jnp.float32),
            jax.ShapeDtypeStruct((NW * NSEG * LANES,), jnp.float32),
        ),
        mesh=mesh,
        scratch_types=[
            pltpu.VMEM((T, D), jnp.float32),
            pltpu.VMEM((T, D), jnp.float32),
            pltpu.VMEM((NCH, C), jnp.int32),
            pltpu.VMEM((NCH, C), jnp.int32),
            pltpu.VMEM((SEG_PER_W, D), jnp.float32),
            pltpu.VMEM((NSEG * LANES,), jnp.float32),
            pltpu.VMEM_SHARED((NSEG, D), jnp.float32),
            pltpu.SemaphoreType.DMA,
            pltpu.SemaphoreType.DMA,
            pltpu.SemaphoreType.DMA,
            pltpu.SemaphoreType.DMA,
        ],
    )
    sums, cnts = f(x, batch)
    sums = sums.reshape(NC, NSEG, D)
    cnts = cnts.reshape(NW, NSEG, LANES)
    return pl.pallas_call(
        _combine_body,
        out_shape=jax.ShapeDtypeStruct((NSEG, D), jnp.float32),
    )(sums, cnts)


def kernel(x, batch):
    return _pooled(x, batch.astype(jnp.int32))


# speedup vs baseline: 1.0940x; 1.0940x over previous
import jax
import jax.numpy as jnp
from jax import lax
from jax.experimental import pallas as pl
from jax.experimental.pallas import tpu as pltpu
from jax.experimental.pallas import tpu_sc as plsc

N_ROWS = 320000
D = 128
NSEG = 1024
NC = 2
NS = 16
NW = NC * NS
T = 256
C = 128
NCH = T // C
NT = N_ROWS // T
TPW = (NT + NW - 1) // NW
LANES = 16
SEG_PER_W = NSEG // NS


def _body(x_hbm, b_hbm, sums_hbm, cnts_hbm, xbuf0, xbuf1, idv0, idv1,
          onesb, zbuf, zc, acc_sh, cnt_sh, sx0, sx1, si0, si1, sc0, sc1):
    cid = lax.axis_index("c")
    sid = lax.axis_index("s")
    wid = cid * NS + sid

    zf = jnp.zeros((LANES,), jnp.float32)
    for r in range(SEG_PER_W):
        for j in range(D // LANES):
            zbuf[r, pl.ds(j * LANES, LANES)] = zf
        zc[r, pl.ds(0, LANES)] = zf
    for r in range(C):
        onesb[r, pl.ds(0, LANES)] = zf + 1.0
    my0 = sid * SEG_PER_W
    pltpu.sync_copy(zbuf, acc_sh.at[pl.ds(my0, SEG_PER_W), :])
    pltpu.sync_copy(zc, cnt_sh.at[pl.ds(my0, SEG_PER_W), :])
    plsc.subcore_barrier()

    nt = lax.div(NT - 1 - wid, NW) + 1

    xbufs = (xbuf0, xbuf1)
    idvs = (idv0, idv1)
    sxs = (sx0, sx1)
    sis = (si0, si1)

    def start(t, b):
        base = pl.multiple_of((t * NW + wid) * T, T)
        pltpu.async_copy(x_hbm.at[pl.ds(base, T), :], xbufs[b], sxs[b])
        for ch in range(NCH):
            pltpu.async_copy(b_hbm.at[pl.ds(base + ch * C, C)],
                             idvs[b].at[ch], sis[b])

    def wait(b):
        pltpu.make_async_copy(x_hbm.at[pl.ds(0, T), :], xbufs[b],
                              sxs[b]).wait()
        for ch in range(NCH):
            pltpu.make_async_copy(b_hbm.at[pl.ds(0, C)], idvs[b].at[ch],
                                  sis[b]).wait()

    scs = (sc0, sc1)

    def process(b):
        for ch in range(NCH):
            pltpu.async_copy(xbufs[b].at[pl.ds(ch * C, C), :],
                             acc_sh.at[idvs[b].at[ch]], scs[b], add=True)
            pltpu.async_copy(onesb, cnt_sh.at[idvs[b].at[ch]], scs[b],
                             add=True)
        pltpu.make_async_copy(x_hbm.at[pl.ds(0, T), :], xbufs[b],
                              scs[b]).wait()
        pltpu.make_async_copy(x_hbm.at[pl.ds(0, NCH * C // 8), :],
                              xbufs[b].at[pl.ds(0, NCH * C // 8), :],
                              scs[b]).wait()

    @pl.when(nt > 0)
    def _():
        start(0, 0)

    @pl.when(nt > 1)
    def _():
        start(1, 1)

    def pair_body(p, _):
        for b in range(2):
            t = p * 2 + b

            @pl.when(t < nt)
            def _():
                wait(b)
                process(b)

                @pl.when(t + 2 < nt)
                def _():
                    start(t + 2, b)

        return 0

    lax.fori_loop(0, lax.div(nt + 1, 2), pair_body, 0)

    plsc.subcore_barrier()
    out_row = cid * NSEG + my0
    pltpu.sync_copy(acc_sh.at[pl.ds(my0, SEG_PER_W), :], zbuf)
    pltpu.sync_copy(zbuf, sums_hbm.at[pl.ds(out_row, SEG_PER_W), :])
    pltpu.sync_copy(cnt_sh.at[pl.ds(my0, SEG_PER_W), :], zc)
    pltpu.sync_copy(zc, cnts_hbm.at[pl.ds(out_row, SEG_PER_W), :])


def _combine_body(s_ref, c_ref, o_ref):
    s = s_ref[0] + s_ref[1]
    c = jnp.maximum(c_ref[0] + c_ref[1], 1.0)
    o_ref[...] = s / c[:, :1]


@jax.jit
def _pooled(x, batch):
    mesh = plsc.VectorSubcoreMesh(core_axis_name="c", subcore_axis_name="s")
    f = pl.kernel(
        _body,
        out_type=(
            jax.ShapeDtypeStruct((NC * NSEG, D), jnp.float32),
            jax.ShapeDtypeStruct((NC * NSEG, LANES), jnp.float32),
        ),
        mesh=mesh,
        scratch_types=[
            pltpu.VMEM((T, D), jnp.float32),
            pltpu.VMEM((T, D), jnp.float32),
            pltpu.VMEM((NCH, C), jnp.int32),
            pltpu.VMEM((NCH, C), jnp.int32),
            pltpu.VMEM((C, LANES), jnp.float32),
            pltpu.VMEM((SEG_PER_W, D), jnp.float32),
            pltpu.VMEM((SEG_PER_W, LANES), jnp.float32),
            pltpu.VMEM_SHARED((NSEG, D), jnp.float32),
            pltpu.VMEM_SHARED((NSEG, LANES), jnp.float32),
            pltpu.SemaphoreType.DMA,
            pltpu.SemaphoreType.DMA,
            pltpu.SemaphoreType.DMA,
            pltpu.SemaphoreType.DMA,
            pltpu.SemaphoreType.DMA,
            pltpu.SemaphoreType.DMA,
        ],
    )
    sums, cnts = f(x, batch)
    sums = sums.reshape(NC, NSEG, D)
    cnts = cnts.reshape(NC, NSEG, LANES)
    return pl.pallas_call(
        _combine_body,
        out_shape=jax.ShapeDtypeStruct((NSEG, D), jnp.float32),
    )(sums, cnts)


def kernel(x, batch):
    return _pooled(x, batch.astype(jnp.int32))
